# baseline (speedup 1.0000x reference)
import functools

import jax
import jax.numpy as jnp
from jax import lax
from jax.experimental import pallas as pl
from jax.experimental.pallas import tpu as pltpu
from jax.experimental.pallas import tpu_sc as plsc

K = 8192
D = 64
B, H, W = 64, 32, 32
N = B * H * W
PER_BATCH = H * W

_sc_info = plsc.get_sparse_core_info()
_NC = _sc_info.num_cores
_NS = _sc_info.num_subcores
_NW = _NC * _NS
_PER_W = N // _NW
_BATCH_PER_W = _PER_W // PER_BATCH
_ROWS_PER_TILE = K // _NS


@functools.partial(
    pl.kernel,
    out_type=(
        jax.ShapeDtypeStruct((N,), jnp.float32),
        jax.ShapeDtypeStruct((N,), jnp.float32),
        jax.ShapeDtypeStruct((B,), jnp.float32),
        jax.ShapeDtypeStruct((B,), jnp.float32),
        jax.ShapeDtypeStruct((2, K), jnp.float32),
    ),
    mesh=plsc.VectorSubcoreMesh(core_axis_name="c", subcore_axis_name="s"),
    compiler_params=pltpu.CompilerParams(
        use_tc_tiling_on_sc=False, needs_layout_passes=False),
    scratch_types=[
        pltpu.VMEM((_ROWS_PER_TILE * D,), jnp.float32),
        pltpu.VMEM((2 * D,), jnp.float32),
        pltpu.VMEM((16,), jnp.float32),
        pltpu.VMEM((_ROWS_PER_TILE,), jnp.float32),
        pltpu.VMEM((_ROWS_PER_TILE,), jnp.float32),
        pltpu.VMEM((K,), jnp.float32),
        pltpu.VMEM((K,), jnp.float32),
        pltpu.VMEM((_PER_W,), jnp.int32),
        pltpu.VMEM((_PER_W,), jnp.float32),
        pltpu.VMEM((_PER_W,), jnp.float32),
        pltpu.VMEM((16,), jnp.float32),
        pltpu.VMEM((16, 16), jnp.float32),
        pltpu.VMEM((32,), jnp.float32),
        pltpu.VMEM((32,), jnp.float32),
        pltpu.VMEM_SHARED((16, 16), jnp.float32),
        pltpu.SemaphoreType.DMA,
        pltpu.SemaphoreType.DMA,
        pltpu.SemaphoreType.DMA,
    ],
)
def _sc_all(cb_hbm, w_hbm, b_hbm, z_hbm, endo_hbm, nuc_hbm, alea_hbm,
            epis_hbm, tab_hbm, cb_v, w_v, b_v, my_t0, my_t1, t0_v, t1_v,
            idx_v, e_v, n_v, m_v, coll_v, av_v, ev_v, m_shared,
            sem0, sem1, sem2):
    cid = lax.axis_index("c")
    sid = lax.axis_index("s")
    wid = cid * _NS + sid
    base = wid * _PER_W
    cp_idx = pltpu.async_copy(z_hbm.at[pl.ds(base, _PER_W)], idx_v, sem0)
    row0 = sid * _ROWS_PER_TILE
    cp_cb = pltpu.async_copy(cb_hbm.at[pl.ds(row0 * D, _ROWS_PER_TILE * D)],
                             cb_v, sem1)
    cp_w = pltpu.async_copy(w_hbm, w_v, sem2)
    cp_w.wait()
    cp_b = pltpu.async_copy(b_hbm, b_v, sem2)
    cp_b.wait()
    cp_cb.wait()

    lane = lax.iota(jnp.int32, 16)
    zero = jnp.zeros((16,), jnp.float32)

    b_vec = b_v[...]
    b0 = b_vec[0]
    b1 = b_vec[1]
    w_vecs = [w_v[pl.ds(k * 16, 16)] for k in range(2 * D // 16)]
    w0s = [w_vecs[(2 * d) // 16][(2 * d) % 16] for d in range(D)]
    w1s = [w_vecs[(2 * d + 1) // 16][(2 * d + 1) % 16] for d in range(D)]
    lane64 = lane * D
    _N_GROUPS = _ROWS_PER_TILE // 16

    def tbody(g, carry):
        rows64 = lane64 + g * (16 * D)
        acc0 = [zero, zero]
        acc1 = [zero, zero]
        for d in range(D):
            col = plsc.load_gather(cb_v, [rows64 + d])
            acc0[d % 2] = acc0[d % 2] + col * w0s[d]
            acc1[d % 2] = acc1[d % 2] + col * w1s[d]
        t0 = acc0[0] + acc0[1] + b0
        t1 = acc1[0] + acc1[1] + b1
        my_t0[pl.ds(g * 16, 16)] = 1.0 / (1.0 + jnp.exp(-t0))
        my_t1[pl.ds(g * 16, 16)] = 1.0 / (1.0 + jnp.exp(-t1))
        return carry

    lax.fori_loop(0, _N_GROUPS, tbody, 0)

    cp_s0 = pltpu.async_copy(
        my_t0, tab_hbm.at[0, pl.ds(row0, _ROWS_PER_TILE)], sem1)
    cp_s1 = pltpu.async_copy(
        my_t1, tab_hbm.at[1, pl.ds(row0, _ROWS_PER_TILE)], sem2)
    cp_s0.wait()
    cp_s1.wait()
    plsc.subcore_barrier()
    cp_t0 = pltpu.async_copy(tab_hbm.at[0], t0_v, sem1)
    cp_t1 = pltpu.async_copy(tab_hbm.at[1], t1_v, sem2)
    cp_t0.wait()
    cp_t1.wait()

    m_row = zero
    _ILP = 4
    for b in range(_BATCH_PER_W):
        acc_e = [zero] * _ILP
        acc_n = [zero] * _ILP
        for i in range(PER_BATCH // 16):
            off = b * PER_BATCH + i * 16
            idx = idx_v[pl.ds(off, 16)]
            e = plsc.load_gather(t0_v, [idx])
            n = plsc.load_gather(t1_v, [idx])
            e_v[pl.ds(off, 16)] = e
            n_v[pl.ds(off, 16)] = n
            acc_e[i % _ILP] = acc_e[i % _ILP] + e
            acc_n[i % _ILP] = acc_n[i % _ILP] + n
        mean_e = jnp.sum(sum(acc_e[1:], acc_e[0])) * (1.0 / PER_BATCH)
        mean_n = jnp.sum(sum(acc_n[1:], acc_n[0])) * (1.0 / PER_BATCH)
        m_row = m_row + jnp.where(lane == b, mean_e, 0.0)
        m_row = m_row + jnp.where(lane == _BATCH_PER_W + b, mean_n, 0.0)

    m_v[...] = m_row
    cp_e = pltpu.async_copy(e_v, endo_hbm.at[pl.ds(base, _PER_W)], sem0)
    cp_n = pltpu.async_copy(n_v, nuc_hbm.at[pl.ds(base, _PER_W)], sem1)

    pltpu.sync_copy(m_v, m_shared.at[sid])
    plsc.subcore_barrier()

    @pl.when(sid == 0)
    def _collect():
        pltpu.sync_copy(m_shared, coll_v)
        for g in range(2):
            jj = lane + 16 * g
            row = jj // 2
            col = jj % 2
            av = plsc.load_gather(coll_v, [row, col])
            ev = plsc.load_gather(coll_v, [row, col + 2])
            av_v[pl.ds(16 * g, 16)] = av
            ev_v[pl.ds(16 * g, 16)] = ev
        pltpu.sync_copy(av_v, alea_hbm.at[pl.ds(32 * cid, 32)])
        pltpu.sync_copy(ev_v, epis_hbm.at[pl.ds(32 * cid, 32)])

    cp_e.wait()
    cp_n.wait()


def kernel(z, codebook, lin_w, lin_b):
    z_flat = z.reshape(-1).astype(jnp.int32)
    cb_flat = codebook.reshape(-1)
    w_flat = lin_w.astype(jnp.float32).reshape(-1)
    b_pad = jnp.pad(lin_b.astype(jnp.float32), (0, 14))
    e_flat, n_flat, alea, epis, _ = _sc_all(cb_flat, w_flat, b_pad, z_flat)
    endosome = e_flat.reshape(B, 1, H, W)
    nuclear = n_flat.reshape(B, 1, H, W)
    return (endosome, nuclear, alea.reshape(B, 1), epis.reshape(B, 1))


# speedup vs baseline: 1.3624x; 1.3624x over previous
import functools

import jax
import jax.numpy as jnp
from jax import lax
from jax.experimental import pallas as pl
from jax.experimental.pallas import tpu as pltpu
from jax.experimental.pallas import tpu_sc as plsc

K = 8192
D = 64
B, H, W = 64, 32, 32
N = B * H * W
PER_BATCH = H * W


def _table_body(cb_ref, w_ref, b_ref, out_ref):
    t = lax.dot_general(
        w_ref[...], cb_ref[...],
        dimension_numbers=(((0,), (1,)), ((), ())),
        preferred_element_type=jnp.float32,
    )
    ch = lax.broadcasted_iota(jnp.int32, (2, K), 0)
    bias = jnp.where(ch == 0, b_ref[0], b_ref[1])
    out_ref[...] = jax.nn.sigmoid(t + bias)


def _make_table(codebook, lin_w, lin_b):
    return pl.pallas_call(
        _table_body,
        out_shape=jax.ShapeDtypeStruct((2, K), jnp.float32),
        in_specs=[
            pl.BlockSpec(memory_space=pltpu.VMEM),
            pl.BlockSpec(memory_space=pltpu.VMEM),
            pl.BlockSpec(memory_space=pltpu.SMEM),
        ],
        out_specs=pl.BlockSpec(memory_space=pltpu.VMEM),
    )(codebook, lin_w, lin_b)


_sc_info = plsc.get_sparse_core_info()
_NC = _sc_info.num_cores
_NS = _sc_info.num_subcores
_NW = _NC * _NS
_PER_W = N // _NW
_BATCH_PER_W = _PER_W // PER_BATCH


@functools.partial(
    pl.kernel,
    out_type=(
        jax.ShapeDtypeStruct((N,), jnp.float32),
        jax.ShapeDtypeStruct((N,), jnp.float32),
        jax.ShapeDtypeStruct((B,), jnp.float32),
        jax.ShapeDtypeStruct((B,), jnp.float32),
    ),
    mesh=plsc.VectorSubcoreMesh(core_axis_name="c", subcore_axis_name="s"),
    compiler_params=pltpu.CompilerParams(
        use_tc_tiling_on_sc=False, needs_layout_passes=False),
    scratch_types=[
        pltpu.VMEM((2 * K,), jnp.float32),
        pltpu.VMEM((_PER_W,), jnp.int32),
        pltpu.VMEM((_PER_W,), jnp.float32),
        pltpu.VMEM((_PER_W,), jnp.float32),
        pltpu.VMEM((16,), jnp.float32),
        pltpu.VMEM((16, 16), jnp.float32),
        pltpu.VMEM((32,), jnp.float32),
        pltpu.VMEM((32,), jnp.float32),
        pltpu.VMEM_SHARED((16, 16), jnp.float32),
        pltpu.SemaphoreType.DMA,
        pltpu.SemaphoreType.DMA,
        pltpu.SemaphoreType.DMA,
    ],
)
def _sc_gather(table_hbm, z_hbm, endo_hbm, nuc_hbm, alea_hbm, epis_hbm,
               t_v, idx_v, e_v, n_v, m_v, coll_v, av_v, ev_v,
               m_shared, sem0, sem1, sem2):
    cid = lax.axis_index("c")
    sid = lax.axis_index("s")
    wid = cid * _NS + sid
    base = wid * _PER_W
    cp_idx = pltpu.async_copy(z_hbm.at[pl.ds(base, _PER_W)], idx_v, sem0)
    cp_t = pltpu.async_copy(table_hbm, t_v, sem1)
    cp_idx.wait()
    cp_t.wait()

    zero = jnp.zeros((16,), jnp.float32)
    lane = lax.iota(jnp.int32, 16)
    m_row = zero
    _ILP = 4
    for b in range(_BATCH_PER_W):
        acc_e = [zero] * _ILP
        acc_n = [zero] * _ILP
        for i in range(PER_BATCH // 16):
            off = b * PER_BATCH + i * 16
            idx = idx_v[pl.ds(off, 16)]
            e = plsc.load_gather(t_v, [idx])
            n = plsc.load_gather(t_v, [idx + K])
            e_v[pl.ds(off, 16)] = e
            n_v[pl.ds(off, 16)] = n
            acc_e[i % _ILP] = acc_e[i % _ILP] + e
            acc_n[i % _ILP] = acc_n[i % _ILP] + n
        mean_e = jnp.sum(sum(acc_e[1:], acc_e[0])) * (1.0 / PER_BATCH)
        mean_n = jnp.sum(sum(acc_n[1:], acc_n[0])) * (1.0 / PER_BATCH)
        m_row = m_row + jnp.where(lane == b, mean_e, 0.0)
        m_row = m_row + jnp.where(lane == _BATCH_PER_W + b, mean_n, 0.0)

    m_v[...] = m_row
    cp_e = pltpu.async_copy(e_v, endo_hbm.at[pl.ds(base, _PER_W)], sem0)
    cp_n = pltpu.async_copy(n_v, nuc_hbm.at[pl.ds(base, _PER_W)], sem1)
    pltpu.sync_copy(m_v, m_shared.at[sid])
    plsc.subcore_barrier()
    @pl.when(sid == 0)
    def _collect():
        pltpu.sync_copy(m_shared, coll_v)
        for g in range(2):
            jj = lane + 16 * g
            row = jj // 2
            col = jj % 2
            av = plsc.load_gather(coll_v, [row, col])
            ev = plsc.load_gather(coll_v, [row, col + 2])
            av_v[pl.ds(16 * g, 16)] = av
            ev_v[pl.ds(16 * g, 16)] = ev
        pltpu.sync_copy(av_v, alea_hbm.at[pl.ds(32 * cid, 32)])
        pltpu.sync_copy(ev_v, epis_hbm.at[pl.ds(32 * cid, 32)])

    cp_e.wait()
    cp_n.wait()


def kernel(z, codebook, lin_w, lin_b):
    table = _make_table(codebook, lin_w.astype(jnp.float32),
                        lin_b.astype(jnp.float32))
    z_flat = z.reshape(-1).astype(jnp.int32)
    e_flat, n_flat, alea, epis = _sc_gather(table.reshape(-1), z_flat)
    endosome = e_flat.reshape(B, 1, H, W)
    nuclear = n_flat.reshape(B, 1, H, W)
    return (endosome, nuclear, alea.reshape(B, 1), epis.reshape(B, 1))


# speedup vs baseline: 1.3797x; 1.0127x over previous
import functools

import jax
import jax.numpy as jnp
from jax import lax
from jax.experimental import pallas as pl
from jax.experimental.pallas import tpu as pltpu
from jax.experimental.pallas import tpu_sc as plsc

K = 8192
D = 64
B, H, W = 64, 32, 32
N = B * H * W
PER_BATCH = H * W


def _table_body(cb_ref, w_ref, b_ref, out_ref):
    t = lax.dot_general(
        w_ref[...], cb_ref[...],
        dimension_numbers=(((0,), (1,)), ((), ())),
        preferred_element_type=jnp.float32,
    )
    ch = lax.broadcasted_iota(jnp.int32, (2, K), 0)
    bias = jnp.where(ch == 0, b_ref[0], b_ref[1])
    out_ref[...] = jax.nn.sigmoid(t + bias)


def _make_table(codebook, lin_w, lin_b):
    return pl.pallas_call(
        _table_body,
        out_shape=jax.ShapeDtypeStruct((2, K), jnp.float32),
        in_specs=[
            pl.BlockSpec(memory_space=pltpu.VMEM),
            pl.BlockSpec(memory_space=pltpu.VMEM),
            pl.BlockSpec(memory_space=pltpu.SMEM),
        ],
        out_specs=pl.BlockSpec(memory_space=pltpu.VMEM),
    )(codebook, lin_w, lin_b)


_sc_info = plsc.get_sparse_core_info()
_NC = _sc_info.num_cores
_NS = _sc_info.num_subcores
_NW = _NC * _NS
_PER_W = N // _NW
_BATCH_PER_W = _PER_W // PER_BATCH


@functools.partial(
    pl.kernel,
    out_type=(
        jax.ShapeDtypeStruct((N,), jnp.float32),
        jax.ShapeDtypeStruct((N,), jnp.float32),
        jax.ShapeDtypeStruct((B,), jnp.float32),
        jax.ShapeDtypeStruct((B,), jnp.float32),
    ),
    mesh=plsc.VectorSubcoreMesh(core_axis_name="c", subcore_axis_name="s"),
    compiler_params=pltpu.CompilerParams(
        use_tc_tiling_on_sc=False, needs_layout_passes=False),
    scratch_types=[
        pltpu.VMEM((K,), jnp.float32),
        pltpu.VMEM((K,), jnp.float32),
        pltpu.VMEM((_PER_W,), jnp.int32),
        pltpu.VMEM((_PER_W,), jnp.float32),
        pltpu.VMEM((_PER_W,), jnp.float32),
        pltpu.VMEM((16,), jnp.float32),
        pltpu.VMEM((16, 16), jnp.float32),
        pltpu.VMEM((32,), jnp.float32),
        pltpu.VMEM((32,), jnp.float32),
        pltpu.VMEM_SHARED((16, 16), jnp.float32),
        pltpu.SemaphoreType.DMA,
        pltpu.SemaphoreType.DMA,
        pltpu.SemaphoreType.DMA,
    ],
)
def _sc_gather(table_hbm, z_hbm, endo_hbm, nuc_hbm, alea_hbm, epis_hbm,
               t0_v, t1_v, idx_v, e_v, n_v, m_v, coll_v, av_v, ev_v,
               m_shared, sem0, sem1, sem2):
    cid = lax.axis_index("c")
    sid = lax.axis_index("s")
    wid = cid * _NS + sid
    base = wid * _PER_W
    cp_idx = pltpu.async_copy(z_hbm.at[pl.ds(base, _PER_W)], idx_v, sem0)
    cp_t0 = pltpu.async_copy(table_hbm.at[0], t0_v, sem1)
    cp_t1 = pltpu.async_copy(table_hbm.at[1], t1_v, sem2)
    cp_idx.wait()
    cp_t0.wait()
    cp_t1.wait()

    zero = jnp.zeros((16,), jnp.float32)
    lane = lax.iota(jnp.int32, 16)
    m_row = zero
    _ILP = 4
    for b in range(_BATCH_PER_W):
        acc_e = [zero] * _ILP
        acc_n = [zero] * _ILP
        for i in range(PER_BATCH // 16):
            off = b * PER_BATCH + i * 16
            idx = idx_v[pl.ds(off, 16)]
            e = plsc.load_gather(t0_v, [idx])
            n = plsc.load_gather(t1_v, [idx])
            e_v[pl.ds(off, 16)] = e
            n_v[pl.ds(off, 16)] = n
            acc_e[i % _ILP] = acc_e[i % _ILP] + e
            acc_n[i % _ILP] = acc_n[i % _ILP] + n
        mean_e = jnp.sum(sum(acc_e[1:], acc_e[0])) * (1.0 / PER_BATCH)
        mean_n = jnp.sum(sum(acc_n[1:], acc_n[0])) * (1.0 / PER_BATCH)
        m_row = m_row + jnp.where(lane == b, mean_e, 0.0)
        m_row = m_row + jnp.where(lane == _BATCH_PER_W + b, mean_n, 0.0)

    m_v[...] = m_row
    cp_e = pltpu.async_copy(e_v, endo_hbm.at[pl.ds(base, _PER_W)], sem0)
    cp_n = pltpu.async_copy(n_v, nuc_hbm.at[pl.ds(base, _PER_W)], sem1)
    pltpu.sync_copy(m_v, m_shared.at[sid])
    plsc.subcore_barrier()
    @pl.when(sid == 0)
    def _collect():
        pltpu.sync_copy(m_shared, coll_v)
        for g in range(2):
            jj = lane + 16 * g
            row = jj // 2
            col = jj % 2
            av = plsc.load_gather(coll_v, [row, col])
            ev = plsc.load_gather(coll_v, [row, col + 2])
            av_v[pl.ds(16 * g, 16)] = av
            ev_v[pl.ds(16 * g, 16)] = ev
        pltpu.sync_copy(av_v, alea_hbm.at[pl.ds(32 * cid, 32)])
        pltpu.sync_copy(ev_v, epis_hbm.at[pl.ds(32 * cid, 32)])

    cp_e.wait()
    cp_n.wait()


def kernel(z, codebook, lin_w, lin_b):
    table = _make_table(codebook, lin_w.astype(jnp.float32),
                        lin_b.astype(jnp.float32))
    z_flat = z.reshape(-1).astype(jnp.int32)
    e_flat, n_flat, alea, epis = _sc_gather(table, z_flat)
    endosome = e_flat.reshape(B, 1, H, W)
    nuclear = n_flat.reshape(B, 1, H, W)
    return (endosome, nuclear, alea.reshape(B, 1), epis.reshape(B, 1))


# speedup vs baseline: 1.3809x; 1.0009x over previous
import functools

import jax
import jax.numpy as jnp
from jax import lax
from jax.experimental import pallas as pl
from jax.experimental.pallas import tpu as pltpu
from jax.experimental.pallas import tpu_sc as plsc

K = 8192
D = 64
B, H, W = 64, 32, 32
N = B * H * W
PER_BATCH = H * W


def _table_body(cb_ref, w_ref, b_ref, out_ref):
    t = lax.dot_general(
        w_ref[...], cb_ref[...],
        dimension_numbers=(((0,), (1,)), ((), ())),
        preferred_element_type=jnp.float32,
    )
    ch = lax.broadcasted_iota(jnp.int32, (2, K), 0)
    bias = jnp.where(ch == 0, b_ref[0], b_ref[1])
    out_ref[...] = jax.nn.sigmoid(t + bias)


def _make_table(codebook, lin_w, lin_b):
    return pl.pallas_call(
        _table_body,
        out_shape=jax.ShapeDtypeStruct((2, K), jnp.float32),
        in_specs=[
            pl.BlockSpec(memory_space=pltpu.VMEM),
            pl.BlockSpec(memory_space=pltpu.VMEM),
            pl.BlockSpec(memory_space=pltpu.SMEM),
        ],
        out_specs=pl.BlockSpec(memory_space=pltpu.VMEM),
    )(codebook, lin_w, lin_b)


try:
    _sc_info = plsc.get_sparse_core_info()
    _NC = _sc_info.num_cores
    _NS = _sc_info.num_subcores
except Exception:
    _NC, _NS = 2, 16
_NW = _NC * _NS
_PER_W = N // _NW
_BATCH_PER_W = _PER_W // PER_BATCH


@functools.cache
def _build_sc_gather():
    decorate = functools.partial(
        pl.kernel,
        out_type=(
            jax.ShapeDtypeStruct((N,), jnp.float32),
            jax.ShapeDtypeStruct((N,), jnp.float32),
            jax.ShapeDtypeStruct((B,), jnp.float32),
            jax.ShapeDtypeStruct((B,), jnp.float32),
        ),
        mesh=plsc.VectorSubcoreMesh(
            core_axis_name="c", subcore_axis_name="s",
            num_cores=_NC, num_subcores=_NS),
        compiler_params=pltpu.CompilerParams(
            use_tc_tiling_on_sc=False, needs_layout_passes=False),
        scratch_types=[
            pltpu.VMEM((K,), jnp.float32),
            pltpu.VMEM((K,), jnp.float32),
            pltpu.VMEM((_PER_W,), jnp.int32),
            pltpu.VMEM((_PER_W,), jnp.float32),
            pltpu.VMEM((_PER_W,), jnp.float32),
            pltpu.VMEM((16,), jnp.float32),
            pltpu.VMEM((16, 16), jnp.float32),
            pltpu.VMEM((32,), jnp.float32),
            pltpu.VMEM((32,), jnp.float32),
            pltpu.VMEM_SHARED((16, 16), jnp.float32),
            pltpu.SemaphoreType.DMA,
            pltpu.SemaphoreType.DMA,
            pltpu.SemaphoreType.DMA,
        ],
    )
    return decorate(_sc_gather_body)


def _sc_gather_body(table_hbm, z_hbm, endo_hbm, nuc_hbm, alea_hbm, epis_hbm,
               t0_v, t1_v, idx_v, e_v, n_v, m_v, coll_v, av_v, ev_v,
               m_shared, sem0, sem1, sem2):
    cid = lax.axis_index("c")
    sid = lax.axis_index("s")
    wid = cid * _NS + sid
    base = wid * _PER_W
    cp_idx = pltpu.async_copy(z_hbm.at[pl.ds(base, _PER_W)], idx_v, sem0)
    cp_t0 = pltpu.async_copy(table_hbm.at[0], t0_v, sem1)
    cp_t1 = pltpu.async_copy(table_hbm.at[1], t1_v, sem2)
    cp_idx.wait()
    cp_t0.wait()
    cp_t1.wait()

    zero = jnp.zeros((16,), jnp.float32)
    lane = lax.iota(jnp.int32, 16)
    m_row = zero
    _ILP = 4
    for b in range(_BATCH_PER_W):
        acc_e = [zero] * _ILP
        acc_n = [zero] * _ILP
        for i in range(PER_BATCH // 16):
            off = b * PER_BATCH + i * 16
            idx = idx_v[pl.ds(off, 16)]
            e = plsc.load_gather(t0_v, [idx])
            n = plsc.load_gather(t1_v, [idx])
            e_v[pl.ds(off, 16)] = e
            n_v[pl.ds(off, 16)] = n
            acc_e[i % _ILP] = acc_e[i % _ILP] + e
            acc_n[i % _ILP] = acc_n[i % _ILP] + n
        mean_e = jnp.sum(sum(acc_e[1:], acc_e[0])) * (1.0 / PER_BATCH)
        mean_n = jnp.sum(sum(acc_n[1:], acc_n[0])) * (1.0 / PER_BATCH)
        m_row = m_row + jnp.where(lane == b, mean_e, 0.0)
        m_row = m_row + jnp.where(lane == _BATCH_PER_W + b, mean_n, 0.0)

    m_v[...] = m_row
    cp_e = pltpu.async_copy(e_v, endo_hbm.at[pl.ds(base, _PER_W)], sem0)
    cp_n = pltpu.async_copy(n_v, nuc_hbm.at[pl.ds(base, _PER_W)], sem1)
    pltpu.sync_copy(m_v, m_shared.at[sid])
    plsc.subcore_barrier()
    @pl.when(sid == 0)
    def _collect():
        pltpu.sync_copy(m_shared, coll_v)
        for g in range(2):
            jj = lane + 16 * g
            row = jj // 2
            col = jj % 2
            av = plsc.load_gather(coll_v, [row, col])
            ev = plsc.load_gather(coll_v, [row, col + 2])
            av_v[pl.ds(16 * g, 16)] = av
            ev_v[pl.ds(16 * g, 16)] = ev
        pltpu.sync_copy(av_v, alea_hbm.at[pl.ds(32 * cid, 32)])
        pltpu.sync_copy(ev_v, epis_hbm.at[pl.ds(32 * cid, 32)])

    cp_e.wait()
    cp_n.wait()


def kernel(z, codebook, lin_w, lin_b):
    table = _make_table(codebook, lin_w.astype(jnp.float32),
                        lin_b.astype(jnp.float32))
    z_flat = z.reshape(-1).astype(jnp.int32)
    e_flat, n_flat, alea, epis = _build_sc_gather()(table, z_flat)
    endosome = e_flat.reshape(B, 1, H, W)
    nuclear = n_flat.reshape(B, 1, H, W)
    return (endosome, nuclear, alea.reshape(B, 1), epis.reshape(B, 1))
